# baseline (speedup 1.0000x reference)
import functools

import jax
import jax.numpy as jnp
from jax import lax
from jax.experimental import pallas as pl
from jax.experimental.pallas import tpu as pltpu
from jax.experimental.pallas import tpu_sc as plsc

N = 10000
NP = 10240
E = 320000
EP = 327680
D = 128
H = 128
G = 128

NC = 2
NS = 16
NW = NC * NS
EPW = EP // NW
CH = 128
NCH = EPW // CH
RPT = NP // NS

BR = 2560
NB = NP // BR

@functools.cache
def _mesh():
    return plsc.VectorSubcoreMesh(core_axis_name="c", subcore_axis_name="s",
                                  num_cores=NC, num_subcores=NS)


def _sc_deg_body(dst_hbm, zdeg_hbm, out_hbm, dstv, onesv, acc_sh, sem):
    c = lax.axis_index("c")
    s = lax.axis_index("s")
    wid = c * NS + s
    pltpu.sync_copy(zdeg_hbm.at[pl.ds(s * RPT, RPT)],
                    acc_sh.at[pl.ds(s * RPT, RPT)])
    pltpu.sync_copy(dst_hbm.at[wid], dstv)
    for i in range(CH // 16):
        onesv[pl.ds(i * 16, 16)] = jnp.ones((16,), jnp.float32)
    plsc.subcore_barrier()

    @pl.loop(0, NCH // 8)
    def _group(g):
        cps = [pltpu.async_copy(onesv, acc_sh.at[dstv.at[g * 8 + j]], sem,
                                add=True)
               for j in range(8)]
        for cp in cps:
            cp.wait()

    plsc.subcore_barrier()
    pltpu.sync_copy(acc_sh.at[pl.ds(s * RPT, RPT)],
                    out_hbm.at[c, pl.ds(s * RPT, RPT)])


def _sc_deg(dstp, zdeg):
    return pl.kernel(
        _sc_deg_body,
        out_type=jax.ShapeDtypeStruct((NC, NP), jnp.float32),
        mesh=_mesh(),
        scratch_types=[
            pltpu.VMEM((NCH, CH), jnp.int32),
            pltpu.VMEM((CH,), jnp.float32),
            pltpu.VMEM_SHARED((NP,), jnp.float32),
            pltpu.SemaphoreType.DMA,
        ],
    )(dstp, zdeg)


GRP = 40
NGRP = NCH // GRP


def _sc_agg_body(src_hbm, dst_hbm, tab_hbm, ztab_hbm, out_hbm,
                 srcv, dstv, rows0, rows1, acc_sh, sem0, sem1):
    c = lax.axis_index("c")
    s = lax.axis_index("s")
    wid = c * NS + s
    @pl.when(c == 0)
    def _():
        pltpu.sync_copy(tab_hbm.at[pl.ds(s * RPT, RPT)],
                        acc_sh.at[pl.ds(s * RPT, RPT)])

    @pl.when(c == 1)
    def _():
        pltpu.sync_copy(ztab_hbm.at[pl.ds(s * RPT, RPT)],
                        acc_sh.at[pl.ds(s * RPT, RPT)])

    plsc.subcore_barrier()

    rows = (rows0, rows1)
    sems = (sem0, sem1)
    for g in range(NGRP):
        pltpu.sync_copy(src_hbm.at[wid, pl.ds(g * GRP, GRP)], srcv)
        pltpu.sync_copy(dst_hbm.at[wid, pl.ds(g * GRP, GRP)], dstv)
        def _fire(k, b):
            pltpu.async_copy(tab_hbm.at[srcv.at[k, pl.ds(0, CH // 2)]],
                             rows[b].at[pl.ds(0, CH // 2)], sems[b])
            pltpu.async_copy(tab_hbm.at[srcv.at[k, pl.ds(CH // 2, CH // 2)]],
                             rows[b].at[pl.ds(CH // 2, CH // 2)], sems[b])

        def _drain(b):
            for _ in range(2):
                pltpu.make_async_copy(
                    tab_hbm.at[srcv.at[0, pl.ds(0, CH // 2)]],
                    rows[b].at[pl.ds(0, CH // 2)], sems[b]).wait()

        for b in range(2):
            _fire(b, b)

        @pl.loop(0, GRP, step=2)
        def _pair(j):
            for b in range(2):
                jj = j + b
                _drain(b)
                pltpu.sync_copy(rows[b], acc_sh.at[dstv.at[jj]], add=True)
                nxt = jj + 2

                @pl.when(nxt < GRP)
                def _():
                    _fire(nxt, b)

    plsc.subcore_barrier()
    pltpu.sync_copy(acc_sh.at[pl.ds(s * RPT, RPT)],
                    out_hbm.at[c, pl.ds(s * RPT, RPT)])


def _sc_agg(srcp, dstp, table, ztab):
    return pl.kernel(
        _sc_agg_body,
        out_type=jax.ShapeDtypeStruct((NC, NP, H), jnp.float32),
        mesh=_mesh(),
        scratch_types=[
            pltpu.VMEM((GRP, CH), jnp.int32),
            pltpu.VMEM((GRP, CH), jnp.int32),
            pltpu.VMEM((CH, H), jnp.float32),
            pltpu.VMEM((CH, H), jnp.float32),
            pltpu.VMEM_SHARED((NP, H), jnp.float32),
            pltpu.SemaphoreType.DMA,
            pltpu.SemaphoreType.DMA,
        ],
    )(srcp, dstp, table, ztab)


def _dot(a, b):
    return jnp.dot(a, b, preferred_element_type=jnp.float32)


def _tc_u1_body(x_ref, w1_ref, b1_ref, dinv_ref, wg_ref, o_ref):
    h0 = _dot(x_ref[...], w1_ref[...]) + b1_ref[...]
    o_ref[...] = _dot(dinv_ref[...] * h0, wg_ref[...])


def _tc_u1(x_p, W1, b1_2d, dinv2d, Wg0):
    return pl.pallas_call(
        _tc_u1_body,
        grid=(NB,),
        in_specs=[
            pl.BlockSpec((BR, D), lambda i: (i, 0)),
            pl.BlockSpec((D, H), lambda i: (0, 0)),
            pl.BlockSpec((1, H), lambda i: (0, 0)),
            pl.BlockSpec((BR, 1), lambda i: (i, 0)),
            pl.BlockSpec((H, H), lambda i: (0, 0)),
        ],
        out_specs=pl.BlockSpec((BR, H), lambda i: (i, 0)),
        out_shape=jax.ShapeDtypeStruct((NP, H), jnp.float32),
    )(x_p, W1, b1_2d, dinv2d, Wg0)


def _tc_deg_body(p_ref, o_ref):
    deg = jnp.sum(p_ref[...], axis=0, keepdims=True) + 1.0
    o_ref[...] = lax.rsqrt(deg)


def _tc_deg(parts):
    return pl.pallas_call(
        _tc_deg_body,
        grid=(NB,),
        in_specs=[pl.BlockSpec((NC, BR), lambda i: (0, i))],
        out_specs=pl.BlockSpec((1, BR), lambda i: (0, i)),
        out_shape=jax.ShapeDtypeStruct((1, NP), jnp.float32),
    )(parts)


def _tc_layer_body(acc_ref, dinv_ref, b_ref, wn_ref, h_ref, un_ref):
    z = acc_ref[0] + acc_ref[1]
    dinv = dinv_ref[...]
    h = jnp.maximum(dinv * z + b_ref[...], 0.0)
    h_ref[...] = h
    un_ref[...] = _dot(dinv * h, wn_ref[...])


def _tc_layer(accp, dinv2d, b_2d, Wn):
    return pl.pallas_call(
        _tc_layer_body,
        grid=(NB,),
        in_specs=[
            pl.BlockSpec((NC, BR, H), lambda i: (0, i, 0)),
            pl.BlockSpec((BR, 1), lambda i: (i, 0)),
            pl.BlockSpec((1, H), lambda i: (0, 0)),
            pl.BlockSpec((H, H), lambda i: (0, 0)),
        ],
        out_specs=[
            pl.BlockSpec((BR, H), lambda i: (i, 0)),
            pl.BlockSpec((BR, H), lambda i: (i, 0)),
        ],
        out_shape=[
            jax.ShapeDtypeStruct((NP, H), jnp.float32),
            jax.ShapeDtypeStruct((NP, H), jnp.float32),
        ],
    )(accp, dinv2d, b_2d, Wn)


def _tc_proj_body(h_ref, w_ref, o_ref):
    o_ref[...] = _dot(h_ref[...], w_ref[...])


def _tc_proj_add_body(h_ref, w_ref, p_ref, o_ref):
    o_ref[...] = _dot(h_ref[...], w_ref[...]) + p_ref[...]


def _tc_proj(h, W, p_prev=None):
    args = (h, W) if p_prev is None else (h, W, p_prev)
    specs = [
        pl.BlockSpec((BR, H), lambda i: (i, 0)),
        pl.BlockSpec((H, H), lambda i: (0, 0)),
    ]
    if p_prev is not None:
        specs.append(pl.BlockSpec((BR, H), lambda i: (i, 0)))
    return pl.pallas_call(
        _tc_proj_body if p_prev is None else _tc_proj_add_body,
        grid=(NB,),
        in_specs=specs,
        out_specs=pl.BlockSpec((BR, H), lambda i: (i, 0)),
        out_shape=jax.ShapeDtypeStruct((NP, H), jnp.float32),
    )(*args)


def _tc_final_body(acc_ref, dinv_ref, b_ref, p_ref,
                   wc_ref, bla_ref, ids_ref, o_ref,
                   sums_ref, cnt_ref):
    i = pl.program_id(0)
    z = acc_ref[0] + acc_ref[1]
    h3 = jnp.maximum(dinv_ref[...] * z + b_ref[...], 0.0)
    hc = p_ref[...] + _dot(h3, wc_ref[...]) + bla_ref[...]
    ids = ids_ref[...]
    mask = (lax.broadcasted_iota(jnp.int32, (G, BR), 0) == ids
            ).astype(jnp.float32)
    ps = _dot(mask, hc)
    pc = jnp.sum(mask, axis=1, keepdims=True)

    @pl.when(i == 0)
    def _():
        sums_ref[...] = ps
        cnt_ref[...] = pc

    @pl.when(i > 0)
    def _():
        sums_ref[...] += ps
        cnt_ref[...] += pc

    @pl.when(i == NB - 1)
    def _():
        o_ref[...] = sums_ref[...] / jnp.maximum(cnt_ref[...], 1.0)


def _tc_final(accp, dinv2d, bg2_2d, p12, Wc, bla_2d, ids2d):
    return pl.pallas_call(
        _tc_final_body,
        grid=(NB,),
        in_specs=[
            pl.BlockSpec((NC, BR, H), lambda i: (0, i, 0)),
            pl.BlockSpec((BR, 1), lambda i: (i, 0)),
            pl.BlockSpec((1, H), lambda i: (0, 0)),
            pl.BlockSpec((BR, H), lambda i: (i, 0)),
            pl.BlockSpec((H, H), lambda i: (0, 0)),
            pl.BlockSpec((1, H), lambda i: (0, 0)),
            pl.BlockSpec((1, BR), lambda i: (0, i)),
        ],
        out_specs=pl.BlockSpec((G, H), lambda i: (0, 0)),
        out_shape=jax.ShapeDtypeStruct((G, H), jnp.float32),
        scratch_shapes=[
            pltpu.VMEM((G, H), jnp.float32),
            pltpu.VMEM((G, 1), jnp.float32),
        ],
    )(accp, dinv2d, bg2_2d, p12, Wc, bla_2d, ids2d)


def kernel(x, edge_index, batch, W1, b1, Wg0, bg0, Wg1, bg1, Wg2, bg2,
           W_la, b_la):
    f32 = jnp.float32
    pad_ids = (jnp.arange(EP - E, dtype=jnp.int32) % (NP - N)) + N
    srcp = jnp.concatenate([edge_index[0].astype(jnp.int32), pad_ids])
    dstp = jnp.concatenate([edge_index[1].astype(jnp.int32), pad_ids])
    srcp3 = srcp.reshape(NW, NCH, CH)
    dstp3 = dstp.reshape(NW, NCH, CH)

    x_p = jnp.pad(x.astype(f32), ((0, NP - N), (0, 0)))
    ids2d = jnp.pad(batch.astype(jnp.int32), (0, NP - N),
                    constant_values=G).reshape(1, NP)
    ztab = jnp.zeros((NP, H), f32)
    zdeg = jnp.zeros((NP,), f32)

    b1_2d = b1.astype(f32).reshape(1, H)
    bg0_2d = bg0.astype(f32).reshape(1, H)
    bg1_2d = bg1.astype(f32).reshape(1, H)
    bg2_2d = bg2.astype(f32).reshape(1, H)
    bla_2d = b_la.astype(f32).reshape(1, H)
    Wa = W_la[0 * H:1 * H].astype(f32)
    Wb = W_la[1 * H:2 * H].astype(f32)
    Wc = W_la[2 * H:3 * H].astype(f32)

    deg_parts = _sc_deg(dstp3, zdeg)
    dinv_row = _tc_deg(deg_parts)
    dinv2d = dinv_row.reshape(NP, 1)

    u1 = _tc_u1(x_p, W1.astype(f32), b1_2d, dinv2d, Wg0.astype(f32))
    acc1 = _sc_agg(srcp3, dstp3, u1, ztab)
    h1, u2 = _tc_layer(acc1, dinv2d, bg0_2d, Wg1.astype(f32))
    acc2 = _sc_agg(srcp3, dstp3, u2, ztab)
    p1 = _tc_proj(h1, Wa)
    h2, u3 = _tc_layer(acc2, dinv2d, bg1_2d, Wg2.astype(f32))
    acc3 = _sc_agg(srcp3, dstp3, u3, ztab)
    p12 = _tc_proj(h2, Wb, p1)
    return _tc_final(acc3, dinv2d, bg2_2d, p12, Wc, bla_2d, ids2d)


# speedup vs baseline: 1.0028x; 1.0028x over previous
import functools

import jax
import jax.numpy as jnp
from jax import lax
from jax.experimental import pallas as pl
from jax.experimental.pallas import tpu as pltpu
from jax.experimental.pallas import tpu_sc as plsc

N = 10000
NP = 10240
E = 320000
EP = 327680
D = 128
H = 128
G = 128

NC = 2
NS = 16
NW = NC * NS
EPW = EP // NW
CH = 128
NCH = EPW // CH
RPT = NP // NS

BR = 2560
NB = NP // BR

@functools.cache
def _mesh():
    return plsc.VectorSubcoreMesh(core_axis_name="c", subcore_axis_name="s",
                                  num_cores=NC, num_subcores=NS)


def _sc_deg_body(dst_hbm, zdeg_hbm, out_hbm, dstv, onesv, acc_sh, sem):
    c = lax.axis_index("c")
    s = lax.axis_index("s")
    wid = c * NS + s
    pltpu.sync_copy(zdeg_hbm.at[pl.ds(s * RPT, RPT)],
                    acc_sh.at[pl.ds(s * RPT, RPT)])
    pltpu.sync_copy(dst_hbm.at[wid], dstv)
    for i in range(CH // 16):
        onesv[pl.ds(i * 16, 16)] = jnp.ones((16,), jnp.float32)
    plsc.subcore_barrier()

    @pl.loop(0, NCH // 8)
    def _group(g):
        cps = [pltpu.async_copy(onesv, acc_sh.at[dstv.at[g * 8 + j]], sem,
                                add=True)
               for j in range(8)]
        for cp in cps:
            cp.wait()

    plsc.subcore_barrier()
    pltpu.sync_copy(acc_sh.at[pl.ds(s * RPT, RPT)],
                    out_hbm.at[c, pl.ds(s * RPT, RPT)])


def _sc_deg(dstp, zdeg):
    return pl.kernel(
        _sc_deg_body,
        out_type=jax.ShapeDtypeStruct((NC, NP), jnp.float32),
        mesh=_mesh(),
        scratch_types=[
            pltpu.VMEM((NCH, CH), jnp.int32),
            pltpu.VMEM((CH,), jnp.float32),
            pltpu.VMEM_SHARED((NP,), jnp.float32),
            pltpu.SemaphoreType.DMA,
        ],
    )(dstp, zdeg)


GRP = 40
NGRP = NCH // GRP


def _sc_agg_body(src_hbm, dst_hbm, tab_hbm, ztab_hbm, out_hbm,
                 srcv, dstv, rows0, rows1, acc_sh, sem0, sem1):
    c = lax.axis_index("c")
    s = lax.axis_index("s")
    wid = c * NS + s
    @pl.when(c == 0)
    def _():
        pltpu.sync_copy(tab_hbm.at[pl.ds(s * RPT, RPT)],
                        acc_sh.at[pl.ds(s * RPT, RPT)])

    @pl.when(c == 1)
    def _():
        pltpu.sync_copy(ztab_hbm.at[pl.ds(s * RPT, RPT)],
                        acc_sh.at[pl.ds(s * RPT, RPT)])

    plsc.subcore_barrier()

    rows = (rows0, rows1)
    sems = (sem0, sem1)
    for g in range(NGRP):
        pltpu.sync_copy(src_hbm.at[wid, pl.ds(g * GRP, GRP)], srcv)
        pltpu.sync_copy(dst_hbm.at[wid, pl.ds(g * GRP, GRP)], dstv)
        def _fire(k, b):
            pltpu.async_copy(tab_hbm.at[srcv.at[k, pl.ds(0, CH // 2)]],
                             rows[b].at[pl.ds(0, CH // 2)], sems[b])
            pltpu.async_copy(tab_hbm.at[srcv.at[k, pl.ds(CH // 2, CH // 2)]],
                             rows[b].at[pl.ds(CH // 2, CH // 2)], sems[b])

        def _drain(b):
            for _ in range(2):
                pltpu.make_async_copy(
                    tab_hbm.at[srcv.at[0, pl.ds(0, CH // 2)]],
                    rows[b].at[pl.ds(0, CH // 2)], sems[b]).wait()

        for b in range(2):
            _fire(b, b)

        @pl.loop(0, GRP, step=2)
        def _pair(j):
            for b in range(2):
                jj = j + b
                _drain(b)
                pltpu.sync_copy(rows[b], acc_sh.at[dstv.at[jj]], add=True)
                nxt = jj + 2

                @pl.when(nxt < GRP)
                def _():
                    _fire(nxt, b)

    plsc.subcore_barrier()
    pltpu.sync_copy(acc_sh.at[pl.ds(s * RPT, RPT)],
                    out_hbm.at[c, pl.ds(s * RPT, RPT)])


def _sc_agg(srcp, dstp, table, ztab):
    return pl.kernel(
        _sc_agg_body,
        out_type=jax.ShapeDtypeStruct((NC, NP, H), jnp.float32),
        mesh=_mesh(),
        scratch_types=[
            pltpu.VMEM((GRP, CH), jnp.int32),
            pltpu.VMEM((GRP, CH), jnp.int32),
            pltpu.VMEM((CH, H), jnp.float32),
            pltpu.VMEM((CH, H), jnp.float32),
            pltpu.VMEM_SHARED((NP, H), jnp.float32),
            pltpu.SemaphoreType.DMA,
            pltpu.SemaphoreType.DMA,
        ],
    )(srcp, dstp, table, ztab)


def _dot(a, b):
    return jnp.dot(a, b, preferred_element_type=jnp.float32)


def _tc_u1_body(x_ref, w1_ref, b1_ref, dinv_ref, wg_ref, o_ref):
    h0 = _dot(x_ref[...], w1_ref[...]) + b1_ref[...]
    o_ref[...] = _dot(dinv_ref[...] * h0, wg_ref[...])


def _tc_u1(x_p, W1, b1_2d, dinv2d, Wg0):
    return pl.pallas_call(
        _tc_u1_body,
        grid=(NB,),
        in_specs=[
            pl.BlockSpec((BR, D), lambda i: (i, 0)),
            pl.BlockSpec((D, H), lambda i: (0, 0)),
            pl.BlockSpec((1, H), lambda i: (0, 0)),
            pl.BlockSpec((BR, 1), lambda i: (i, 0)),
            pl.BlockSpec((H, H), lambda i: (0, 0)),
        ],
        out_specs=pl.BlockSpec((BR, H), lambda i: (i, 0)),
        out_shape=jax.ShapeDtypeStruct((NP, H), jnp.float32),
    )(x_p, W1, b1_2d, dinv2d, Wg0)


def _tc_deg_body(p_ref, o_ref):
    deg = jnp.sum(p_ref[...], axis=0, keepdims=True) + 1.0
    o_ref[...] = lax.rsqrt(deg)


def _tc_deg(parts):
    return pl.pallas_call(
        _tc_deg_body,
        grid=(NB,),
        in_specs=[pl.BlockSpec((NC, BR), lambda i: (0, i))],
        out_specs=pl.BlockSpec((1, BR), lambda i: (0, i)),
        out_shape=jax.ShapeDtypeStruct((1, NP), jnp.float32),
    )(parts)


def _tc_layer_body(acc_ref, dinv_ref, b_ref, wn_ref, h_ref, un_ref):
    z = acc_ref[0] + acc_ref[1]
    dinv = dinv_ref[...]
    h = jnp.maximum(dinv * z + b_ref[...], 0.0)
    h_ref[...] = h
    un_ref[...] = _dot(dinv * h, wn_ref[...])


def _tc_layer(accp, dinv2d, b_2d, Wn):
    return pl.pallas_call(
        _tc_layer_body,
        grid=(NB,),
        in_specs=[
            pl.BlockSpec((NC, BR, H), lambda i: (0, i, 0)),
            pl.BlockSpec((BR, 1), lambda i: (i, 0)),
            pl.BlockSpec((1, H), lambda i: (0, 0)),
            pl.BlockSpec((H, H), lambda i: (0, 0)),
        ],
        out_specs=[
            pl.BlockSpec((BR, H), lambda i: (i, 0)),
            pl.BlockSpec((BR, H), lambda i: (i, 0)),
        ],
        out_shape=[
            jax.ShapeDtypeStruct((NP, H), jnp.float32),
            jax.ShapeDtypeStruct((NP, H), jnp.float32),
        ],
    )(accp, dinv2d, b_2d, Wn)


def _tc_final_body(acc_ref, dinv_ref, b_ref, h1_ref, h2_ref,
                   wa_ref, wb_ref, wc_ref, bla_ref, ids_ref, o_ref,
                   sums_ref, cnt_ref):
    i = pl.program_id(0)
    z = acc_ref[0] + acc_ref[1]
    h3 = jnp.maximum(dinv_ref[...] * z + b_ref[...], 0.0)
    hc = (_dot(h1_ref[...], wa_ref[...]) + _dot(h2_ref[...], wb_ref[...])
          + _dot(h3, wc_ref[...]) + bla_ref[...])
    ids = ids_ref[...]
    mask = (lax.broadcasted_iota(jnp.int32, (G, BR), 0) == ids
            ).astype(jnp.float32)
    ps = _dot(mask, hc)
    pc = jnp.sum(mask, axis=1, keepdims=True)

    @pl.when(i == 0)
    def _():
        sums_ref[...] = ps
        cnt_ref[...] = pc

    @pl.when(i > 0)
    def _():
        sums_ref[...] += ps
        cnt_ref[...] += pc

    @pl.when(i == NB - 1)
    def _():
        o_ref[...] = sums_ref[...] / jnp.maximum(cnt_ref[...], 1.0)


def _tc_final(accp, dinv2d, bg2_2d, h1, h2, Wa, Wb, Wc, bla_2d, ids2d):
    return pl.pallas_call(
        _tc_final_body,
        grid=(NB,),
        in_specs=[
            pl.BlockSpec((NC, BR, H), lambda i: (0, i, 0)),
            pl.BlockSpec((BR, 1), lambda i: (i, 0)),
            pl.BlockSpec((1, H), lambda i: (0, 0)),
            pl.BlockSpec((BR, H), lambda i: (i, 0)),
            pl.BlockSpec((BR, H), lambda i: (i, 0)),
            pl.BlockSpec((H, H), lambda i: (0, 0)),
            pl.BlockSpec((H, H), lambda i: (0, 0)),
            pl.BlockSpec((H, H), lambda i: (0, 0)),
            pl.BlockSpec((1, H), lambda i: (0, 0)),
            pl.BlockSpec((1, BR), lambda i: (0, i)),
        ],
        out_specs=pl.BlockSpec((G, H), lambda i: (0, 0)),
        out_shape=jax.ShapeDtypeStruct((G, H), jnp.float32),
        scratch_shapes=[
            pltpu.VMEM((G, H), jnp.float32),
            pltpu.VMEM((G, 1), jnp.float32),
        ],
    )(accp, dinv2d, bg2_2d, h1, h2, Wa, Wb, Wc, bla_2d, ids2d)


def kernel(x, edge_index, batch, W1, b1, Wg0, bg0, Wg1, bg1, Wg2, bg2,
           W_la, b_la):
    f32 = jnp.float32
    pad_ids = (jnp.arange(EP - E, dtype=jnp.int32) % (NP - N)) + N
    srcp = jnp.concatenate([edge_index[0].astype(jnp.int32), pad_ids])
    dstp = jnp.concatenate([edge_index[1].astype(jnp.int32), pad_ids])
    srcp3 = srcp.reshape(NW, NCH, CH)
    dstp3 = dstp.reshape(NW, NCH, CH)

    x_p = jnp.pad(x.astype(f32), ((0, NP - N), (0, 0)))
    ids2d = jnp.pad(batch.astype(jnp.int32), (0, NP - N),
                    constant_values=G).reshape(1, NP)
    ztab = jnp.zeros((NP, H), f32)
    zdeg = jnp.zeros((NP,), f32)

    b1_2d = b1.astype(f32).reshape(1, H)
    bg0_2d = bg0.astype(f32).reshape(1, H)
    bg1_2d = bg1.astype(f32).reshape(1, H)
    bg2_2d = bg2.astype(f32).reshape(1, H)
    bla_2d = b_la.astype(f32).reshape(1, H)
    Wa = W_la[0 * H:1 * H].astype(f32)
    Wb = W_la[1 * H:2 * H].astype(f32)
    Wc = W_la[2 * H:3 * H].astype(f32)

    deg_parts = _sc_deg(dstp3, zdeg)
    dinv_row = _tc_deg(deg_parts)
    dinv2d = dinv_row.reshape(NP, 1)

    u1 = _tc_u1(x_p, W1.astype(f32), b1_2d, dinv2d, Wg0.astype(f32))
    acc1 = _sc_agg(srcp3, dstp3, u1, ztab)
    h1, u2 = _tc_layer(acc1, dinv2d, bg0_2d, Wg1.astype(f32))
    acc2 = _sc_agg(srcp3, dstp3, u2, ztab)
    h2, u3 = _tc_layer(acc2, dinv2d, bg1_2d, Wg2.astype(f32))
    acc3 = _sc_agg(srcp3, dstp3, u3, ztab)
    return _tc_final(acc3, dinv2d, bg2_2d, h1, h2, Wa, Wb, Wc,
                     bla_2d, ids2d)
